# baseline (speedup 1.0000x reference)
import jax
import jax.numpy as jnp
from jax import lax
from jax.experimental import pallas as pl
from jax.experimental.pallas import tpu as pltpu
from jax.experimental.pallas import tpu_sc as plsc

B = 4
H = W = 512
HW = H * W
NM = 2048
NU = 1024
MP = NM // 4
UP = NU // 4
OFF_MA = 0
OFF_MB = OFF_MA + 8 * NM * 2
OFF_UA = OFF_MB + 8 * NM * 2
OFF_UB = OFF_UA + 8 * NU * 2

_mesh = plsc.VectorSubcoreMesh(core_axis_name="c", subcore_axis_name="s")

_SCRATCH = [
        pltpu.VMEM((2 * MP,), jnp.int32),
        pltpu.VMEM((2 * MP,), jnp.int32),
        pltpu.VMEM((2 * UP,), jnp.int32),
        pltpu.VMEM((2 * UP,), jnp.int32),
        pltpu.VMEM((MP // 128, 128), jnp.int32),
        pltpu.VMEM((MP // 128, 128), jnp.int32),
        pltpu.VMEM((UP // 128, 128), jnp.int32),
        pltpu.VMEM((UP // 128, 128), jnp.int32),
        pltpu.VMEM((MP // 128, 128), jnp.float32),
        pltpu.VMEM((MP // 128, 128), jnp.float32),
        pltpu.VMEM((UP // 128, 128), jnp.float32),
        pltpu.VMEM((UP // 128, 128), jnp.float32),
        pltpu.VMEM((16,), jnp.float32),
        pltpu.SemaphoreType.DMA,
]


def _betti_body(pred_hbm, tgt_hbm, coords_hbm,
                out_hbm,
                cMA, cMB, cUA, cUB, iMA, iMB, iUA, iUB,
                vMA, vMB, vUA, vUB, accbuf, sem):
    c = lax.axis_index("c")
    s = lax.axis_index("s")
    wid = c * 16 + s
    seg = wid // 4
    part = wid % 4
    b = seg % 4
    kind = seg // 4

    moff = seg * (2 * NM) + part * (2 * MP)
    uoff = seg * (2 * NU) + part * (2 * UP)
    h0 = pltpu.async_copy(coords_hbm.at[pl.ds(OFF_MA + moff, 2 * MP)], cMA, sem)
    h1 = pltpu.async_copy(coords_hbm.at[pl.ds(OFF_MB + moff, 2 * MP)], cMB, sem)
    h2 = pltpu.async_copy(coords_hbm.at[pl.ds(OFF_UA + uoff, 2 * UP)], cUA, sem)
    h3 = pltpu.async_copy(coords_hbm.at[pl.ds(OFF_UB + uoff, 2 * UP)], cUB, sem)
    h0.wait(); h1.wait(); h2.wait(); h3.wait()

    base = b * HW

    lanes = lax.iota(jnp.int32, 16)
    idx_y = (lanes * 2) & 15
    idx_x = (lanes * 2 + 1) & 15
    lo = lanes < 8

    def _flat_idx(cbuf, k):
        v0 = cbuf[pl.ds(32 * k, 16)]
        v1 = cbuf[pl.ds(32 * k + 16, 16)]
        y = jnp.where(lo, v0.at[idx_y].get(mode="promise_in_bounds"),
                      v1.at[idx_y].get(mode="promise_in_bounds"))
        x = jnp.where(lo, v0.at[idx_x].get(mode="promise_in_bounds"),
                      v1.at[idx_x].get(mode="promise_in_bounds"))
        return y * W + x + base

    for k in range(MP // 16):
        row, off = k // 8, (k % 8) * 16
        iMA[row, pl.ds(off, 16)] = _flat_idx(cMA, k)
        iMB[row, pl.ds(off, 16)] = _flat_idx(cMB, k)
    for k in range(UP // 16):
        row, off = k // 8, (k % 8) * 16
        iUA[row, pl.ds(off, 16)] = _flat_idx(cUA, k)
        iUB[row, pl.ds(off, 16)] = _flat_idx(cUB, k)

    handles = []
    for ch in range(MP // 128):
        handles.append(pltpu.async_copy(pred_hbm.at[iMA.at[ch]], vMA.at[ch], sem))
        handles.append(pltpu.async_copy(tgt_hbm.at[iMB.at[ch]], vMB.at[ch], sem))

    @pl.when(kind == 0)
    def _():
        hs = []
        for ch in range(UP // 128):
            hs.append(pltpu.async_copy(pred_hbm.at[iUA.at[ch]], vUA.at[ch], sem))
            hs.append(pltpu.async_copy(pred_hbm.at[iUB.at[ch]], vUB.at[ch], sem))
        for h in hs:
            h.wait()

    @pl.when(kind == 1)
    def _():
        hs = []
        for ch in range(UP // 128):
            hs.append(pltpu.async_copy(tgt_hbm.at[iUA.at[ch]], vUA.at[ch], sem))
            hs.append(pltpu.async_copy(tgt_hbm.at[iUB.at[ch]], vUB.at[ch], sem))
        for h in hs:
            h.wait()

    for h in handles:
        h.wait()

    acc_m = jnp.zeros((16,), jnp.float32)
    acc_u = jnp.zeros((16,), jnp.float32)
    for ch in range(MP // 128):
        for k in range(8):
            d = vMA[ch, pl.ds(k * 16, 16)] - vMB[ch, pl.ds(k * 16, 16)]
            acc_m = acc_m + d * d
    for ch in range(UP // 128):
        for k in range(8):
            d = vUA[ch, pl.ds(k * 16, 16)] - vUB[ch, pl.ds(k * 16, 16)]
            acc_u = acc_u + d * d
    part_acc = (acc_m * 2.0 + acc_u) * (1.0 / B)

    ssum = jnp.float32(0.0)
    for i in range(16):
        ssum = ssum + part_acc[i]
    accbuf[...] = jnp.full((16,), ssum, jnp.float32)
    pltpu.sync_copy(accbuf, out_hbm.at[wid])


_betti_sc = pl.kernel(
    _betti_body,
    out_type=jax.ShapeDtypeStruct((32, 16), jnp.float32),
    mesh=_mesh,
    scratch_types=_SCRATCH,
)


def kernel(input, target, pred_matched_birth, pred_matched_death,
           tgt_matched_birth, tgt_matched_death,
           pred_unmatched_birth, pred_unmatched_death,
           tgt_unmatched_birth, tgt_unmatched_death):
    pred = input.reshape(B * HW)
    tgt = target.reshape(B * HW)
    coords = jnp.concatenate([
        pred_matched_birth.reshape(-1), pred_matched_death.reshape(-1),
        tgt_matched_birth.reshape(-1), tgt_matched_death.reshape(-1),
        pred_unmatched_birth.reshape(-1), tgt_unmatched_birth.reshape(-1),
        pred_unmatched_death.reshape(-1), tgt_unmatched_death.reshape(-1),
    ])
    out = _betti_sc(pred, tgt, coords)
    return out[:, 0].sum()


# speedup vs baseline: 1.6353x; 1.6353x over previous
import jax
import jax.numpy as jnp
from jax import lax
from jax.experimental import pallas as pl
from jax.experimental.pallas import tpu as pltpu
from jax.experimental.pallas import tpu_sc as plsc

B = 4
H = W = 512
HW = H * W
NM = 2048
NU = 1024
MP = NM // 4
UP = NU // 4
OFF_MA = 0
OFF_MB = OFF_MA + 8 * NM * 2
OFF_UA = OFF_MB + 8 * NM * 2
OFF_UB = OFF_UA + 8 * NU * 2

_mesh = plsc.VectorSubcoreMesh(core_axis_name="c", subcore_axis_name="s")

_SCRATCH = [
        pltpu.VMEM((2 * MP,), jnp.int32),
        pltpu.VMEM((2 * MP,), jnp.int32),
        pltpu.VMEM((2 * UP,), jnp.int32),
        pltpu.VMEM((2 * UP,), jnp.int32),
        pltpu.VMEM((MP // 128, 128), jnp.int32),
        pltpu.VMEM((MP // 128, 128), jnp.int32),
        pltpu.VMEM((UP // 128, 128), jnp.int32),
        pltpu.VMEM((UP // 128, 128), jnp.int32),
        pltpu.VMEM((MP // 128, 128), jnp.float32),
        pltpu.VMEM((MP // 128, 128), jnp.float32),
        pltpu.VMEM((UP // 128, 128), jnp.float32),
        pltpu.VMEM((UP // 128, 128), jnp.float32),
        pltpu.VMEM((16,), jnp.float32),
        pltpu.SemaphoreType.DMA,
]


def _betti_body(pred_hbm, tgt_hbm, coords_hbm, out_hbm,
                cMA, cMB, cUA, cUB, iMA, iMB, iUA, iUB,
                vMA, vMB, vUA, vUB, accbuf, sem):
    c = lax.axis_index("c")
    s = lax.axis_index("s")
    wid = c * 16 + s
    seg = wid // 4
    part = wid % 4
    b = seg // 2
    kind = seg % 2

    moff = seg * (2 * NM) + part * (2 * MP)
    uoff = seg * (2 * NU) + part * (2 * UP)
    h0 = pltpu.async_copy(coords_hbm.at[pl.ds(OFF_MA + moff, 2 * MP)], cMA, sem)
    h1 = pltpu.async_copy(coords_hbm.at[pl.ds(OFF_MB + moff, 2 * MP)], cMB, sem)
    h2 = pltpu.async_copy(coords_hbm.at[pl.ds(OFF_UA + uoff, 2 * UP)], cUA, sem)
    h3 = pltpu.async_copy(coords_hbm.at[pl.ds(OFF_UB + uoff, 2 * UP)], cUB, sem)
    h0.wait()
    h1.wait()
    h2.wait()
    h3.wait()

    base = b * HW

    handles = []
    for ch in range(MP // 128):
        for k8 in range(8):
            k = ch * 8 + k8
            y = cMA[pl.ds(16 * k, 16)]
            x = cMA[pl.ds(MP + 16 * k, 16)]
            iMA[ch, pl.ds(16 * k8, 16)] = y * W + x + base
        handles.append(pltpu.async_copy(pred_hbm.at[iMA.at[ch]], vMA.at[ch], sem))
        for k8 in range(8):
            k = ch * 8 + k8
            y = cMB[pl.ds(16 * k, 16)]
            x = cMB[pl.ds(MP + 16 * k, 16)]
            iMB[ch, pl.ds(16 * k8, 16)] = y * W + x + base
        handles.append(pltpu.async_copy(tgt_hbm.at[iMB.at[ch]], vMB.at[ch], sem))
    for k in range(UP // 16):
        row, off = k // 8, (k % 8) * 16
        y = cUA[pl.ds(16 * k, 16)]
        x = cUA[pl.ds(UP + 16 * k, 16)]
        iUA[row, pl.ds(off, 16)] = y * W + x + base
        y = cUB[pl.ds(16 * k, 16)]
        x = cUB[pl.ds(UP + 16 * k, 16)]
        iUB[row, pl.ds(off, 16)] = y * W + x + base

    @pl.when(kind == 0)
    def _():
        hs = []
        for ch in range(UP // 128):
            hs.append(pltpu.async_copy(pred_hbm.at[iUA.at[ch]], vUA.at[ch], sem))
            hs.append(pltpu.async_copy(pred_hbm.at[iUB.at[ch]], vUB.at[ch], sem))
        for h in hs:
            h.wait()

    @pl.when(kind == 1)
    def _():
        hs = []
        for ch in range(UP // 128):
            hs.append(pltpu.async_copy(tgt_hbm.at[iUA.at[ch]], vUA.at[ch], sem))
            hs.append(pltpu.async_copy(tgt_hbm.at[iUB.at[ch]], vUB.at[ch], sem))
        for h in hs:
            h.wait()

    for h in handles:
        h.wait()

    acc_m = jnp.zeros((16,), jnp.float32)
    acc_u = jnp.zeros((16,), jnp.float32)
    for ch in range(MP // 128):
        for k in range(8):
            d = vMA[ch, pl.ds(k * 16, 16)] - vMB[ch, pl.ds(k * 16, 16)]
            acc_m = acc_m + d * d
    for ch in range(UP // 128):
        for k in range(8):
            d = vUA[ch, pl.ds(k * 16, 16)] - vUB[ch, pl.ds(k * 16, 16)]
            acc_u = acc_u + d * d
    part_acc = (acc_m * 2.0 + acc_u) * (1.0 / B)

    ssum = jnp.float32(0.0)
    for i in range(16):
        ssum = ssum + part_acc[i]
    accbuf[...] = jnp.full((16,), ssum, jnp.float32)
    pltpu.sync_copy(accbuf, out_hbm.at[wid])


_betti_sc = pl.kernel(
    _betti_body,
    out_type=jax.ShapeDtypeStruct((32, 16), jnp.float32),
    mesh=_mesh,
    scratch_types=_SCRATCH,
)


def kernel(input, target, pred_matched_birth, pred_matched_death,
           tgt_matched_birth, tgt_matched_death,
           pred_unmatched_birth, pred_unmatched_death,
           tgt_unmatched_birth, tgt_unmatched_death):
    pred = input.reshape(B * HW)
    tgt = target.reshape(B * HW)
    def _blk(lhs, rhs, n_pairs):
        a = jnp.stack([lhs, rhs], axis=1)
        a = a.reshape(8, 4, n_pairs // 4, 2)
        return jnp.swapaxes(a, 2, 3).reshape(-1)

    ma = _blk(pred_matched_birth, pred_matched_death, NM)
    mb = _blk(tgt_matched_birth, tgt_matched_death, NM)
    ua = _blk(pred_unmatched_birth, tgt_unmatched_birth, NU)
    ub = _blk(pred_unmatched_death, tgt_unmatched_death, NU)
    coords = jnp.concatenate([ma, mb, ua, ub])
    out = _betti_sc(pred, tgt, coords)
    return out[:, 0].sum()


# speedup vs baseline: 1.6398x; 1.0028x over previous
import jax
import jax.numpy as jnp
from jax import lax
from jax.experimental import pallas as pl
from jax.experimental.pallas import tpu as pltpu
from jax.experimental.pallas import tpu_sc as plsc

B = 4
H = W = 512
HW = H * W
NM = 2048
NU = 1024
MP = NM // 4
UP = NU // 4
OFF_MA = 0
OFF_MB = OFF_MA + 8 * NM * 2
OFF_UA = OFF_MB + 8 * NM * 2
OFF_UB = OFF_UA + 8 * NU * 2

_mesh = plsc.VectorSubcoreMesh(core_axis_name="c", subcore_axis_name="s")

_SCRATCH = [
        pltpu.VMEM((2 * MP,), jnp.int32),
        pltpu.VMEM((2 * MP,), jnp.int32),
        pltpu.VMEM((2 * UP,), jnp.int32),
        pltpu.VMEM((2 * UP,), jnp.int32),
        pltpu.VMEM((MP,), jnp.int32),
        pltpu.VMEM((MP,), jnp.int32),
        pltpu.VMEM((UP,), jnp.int32),
        pltpu.VMEM((UP,), jnp.int32),
        pltpu.VMEM((MP,), jnp.float32),
        pltpu.VMEM((MP,), jnp.float32),
        pltpu.VMEM((UP,), jnp.float32),
        pltpu.VMEM((UP,), jnp.float32),
        pltpu.VMEM((16,), jnp.float32),
        pltpu.SemaphoreType.DMA,
]


def _betti_body(pred_hbm, tgt_hbm, coords_hbm, out_hbm,
                cMA, cMB, cUA, cUB, iMA, iMB, iUA, iUB,
                vMA, vMB, vUA, vUB, accbuf, sem):
    c = lax.axis_index("c")
    s = lax.axis_index("s")
    wid = c * 16 + s
    seg = wid // 4
    part = wid % 4
    b = seg // 2
    kind = seg % 2

    moff = seg * (2 * NM) + part * (2 * MP)
    uoff = seg * (2 * NU) + part * (2 * UP)
    h0 = pltpu.async_copy(coords_hbm.at[pl.ds(OFF_MA + moff, 2 * MP)], cMA, sem)
    h1 = pltpu.async_copy(coords_hbm.at[pl.ds(OFF_MB + moff, 2 * MP)], cMB, sem)
    h2 = pltpu.async_copy(coords_hbm.at[pl.ds(OFF_UA + uoff, 2 * UP)], cUA, sem)
    h3 = pltpu.async_copy(coords_hbm.at[pl.ds(OFF_UB + uoff, 2 * UP)], cUB, sem)
    h0.wait()
    h1.wait()
    h2.wait()
    h3.wait()

    base = b * HW

    def _build_m(k, _):
        o = 16 * k
        iMA[pl.ds(o, 16)] = cMA[pl.ds(o, 16)] * W + cMA[pl.ds(MP + o, 16)] + base
        iMB[pl.ds(o, 16)] = cMB[pl.ds(o, 16)] * W + cMB[pl.ds(MP + o, 16)] + base
        return _

    def _build_u(k, _):
        o = 16 * k
        iUA[pl.ds(o, 16)] = cUA[pl.ds(o, 16)] * W + cUA[pl.ds(UP + o, 16)] + base
        iUB[pl.ds(o, 16)] = cUB[pl.ds(o, 16)] * W + cUB[pl.ds(UP + o, 16)] + base
        return _

    lax.fori_loop(0, MP // 16, _build_m, 0)
    lax.fori_loop(0, UP // 16, _build_u, 0)

    handles = []
    for ch in range(MP // 128):
        handles.append(pltpu.async_copy(
            pred_hbm.at[iMA.at[pl.ds(128 * ch, 128)]],
            vMA.at[pl.ds(128 * ch, 128)], sem))
        handles.append(pltpu.async_copy(
            tgt_hbm.at[iMB.at[pl.ds(128 * ch, 128)]],
            vMB.at[pl.ds(128 * ch, 128)], sem))

    @pl.when(kind == 0)
    def _():
        hs = []
        for ch in range(UP // 128):
            hs.append(pltpu.async_copy(
                pred_hbm.at[iUA.at[pl.ds(128 * ch, 128)]],
                vUA.at[pl.ds(128 * ch, 128)], sem))
            hs.append(pltpu.async_copy(
                pred_hbm.at[iUB.at[pl.ds(128 * ch, 128)]],
                vUB.at[pl.ds(128 * ch, 128)], sem))
        for h in hs:
            h.wait()

    @pl.when(kind == 1)
    def _():
        hs = []
        for ch in range(UP // 128):
            hs.append(pltpu.async_copy(
                tgt_hbm.at[iUA.at[pl.ds(128 * ch, 128)]],
                vUA.at[pl.ds(128 * ch, 128)], sem))
            hs.append(pltpu.async_copy(
                tgt_hbm.at[iUB.at[pl.ds(128 * ch, 128)]],
                vUB.at[pl.ds(128 * ch, 128)], sem))
        for h in hs:
            h.wait()

    for h in handles:
        h.wait()

    def _acc_m(k, a):
        o = 16 * k
        d = vMA[pl.ds(o, 16)] - vMB[pl.ds(o, 16)]
        return a + d * d

    def _acc_u(k, a):
        o = 16 * k
        d = vUA[pl.ds(o, 16)] - vUB[pl.ds(o, 16)]
        return a + d * d

    acc_m = lax.fori_loop(0, MP // 16, _acc_m, jnp.zeros((16,), jnp.float32))
    acc_u = lax.fori_loop(0, UP // 16, _acc_u, jnp.zeros((16,), jnp.float32))
    part_acc = (acc_m * 2.0 + acc_u) * (1.0 / B)

    ssum = jnp.float32(0.0)
    for i in range(16):
        ssum = ssum + part_acc[i]
    accbuf[...] = jnp.full((16,), ssum, jnp.float32)
    pltpu.sync_copy(accbuf, out_hbm.at[wid])


_betti_sc = pl.kernel(
    _betti_body,
    out_type=jax.ShapeDtypeStruct((32, 16), jnp.float32),
    mesh=_mesh,
    scratch_types=_SCRATCH,
)


def kernel(input, target, pred_matched_birth, pred_matched_death,
           tgt_matched_birth, tgt_matched_death,
           pred_unmatched_birth, pred_unmatched_death,
           tgt_unmatched_birth, tgt_unmatched_death):
    pred = input.reshape(B * HW)
    tgt = target.reshape(B * HW)
    def _blk(lhs, rhs, n_pairs):
        a = jnp.stack([lhs, rhs], axis=1)
        a = a.reshape(8, 4, n_pairs // 4, 2)
        return jnp.swapaxes(a, 2, 3).reshape(-1)

    ma = _blk(pred_matched_birth, pred_matched_death, NM)
    mb = _blk(tgt_matched_birth, tgt_matched_death, NM)
    ua = _blk(pred_unmatched_birth, tgt_unmatched_birth, NU)
    ub = _blk(pred_unmatched_death, tgt_unmatched_death, NU)
    coords = jnp.concatenate([ma, mb, ua, ub])
    out = _betti_sc(pred, tgt, coords)
    return out[:, 0].sum()
